# Initial kernel scaffold; baseline (speedup 1.0000x reference)
import jax
import jax.numpy as jnp
from jax.experimental import pallas as pl


def kernel(y_true, y_pred, theta):
    raise NotImplementedError("write your pallas kernel here")


# speedup vs baseline: 3.0998x; 3.0998x over previous
import functools

import jax
import jax.numpy as jnp
from jax import lax
from jax.experimental import pallas as pl
from jax.experimental.pallas import tpu as pltpu
from jax.experimental.pallas import tpu_sc as plsc

BAG = 64
CLS = 128
N = 16384
NC = 2
NS = 16
NW = NC * NS
RW = N // NW
HB = 128
LG = CLS // 16


def _sc_body(yt_hbm, yp_hbm, psums_hbm, phist_hbm,
             idx_v, rows_v, acc_v, hist_v):
    c = lax.axis_index("c")
    s = lax.axis_index("s")
    wid = s * NC + c

    pltpu.sync_copy(yt_hbm.at[wid], idx_v)
    pltpu.sync_copy(yp_hbm.at[pl.ds(wid * RW, RW)], rows_v)

    z16 = jnp.zeros((16,), jnp.float32)

    def zero_row(i, _):
        for q in range(LG):
            acc_v[i, pl.ds(q * 16, 16)] = z16
        return 0

    lax.fori_loop(0, BAG, zero_row, 0, unroll=4)

    for q in range(HB // 16):
        hist_v[pl.ds(q * 16, 16)] = z16
    ones = jnp.ones((16,), jnp.float32)
    for k in range(RW // 16):
        ids16 = idx_v[pl.ds(k * 16, 16)]
        plsc.addupdate_scatter(hist_v, [ids16], ones)

    def chunk_add(k, _):
        base = k * 16
        ids16 = idx_v[pl.ds(base, 16)]
        for l in range(16):
            b = ids16[l]
            for q in range(LG):
                plsc.addupdate(acc_v.at[b, pl.ds(q * 16, 16)],
                               rows_v[base + l, pl.ds(q * 16, 16)])
        return 0

    lax.fori_loop(0, RW // 16, chunk_add, 0)

    pltpu.sync_copy(acc_v, psums_hbm.at[wid])
    pltpu.sync_copy(hist_v, phist_hbm.at[wid])


def _sc_call(yt2, y_pred):
    call = functools.partial(
        pl.kernel,
        out_type=[
            jax.ShapeDtypeStruct((NW, BAG, CLS), jnp.float32),
            jax.ShapeDtypeStruct((NW, HB), jnp.float32),
        ],
        mesh=plsc.VectorSubcoreMesh(core_axis_name="c", subcore_axis_name="s",
                                    num_cores=NC, num_subcores=NS),
        scratch_types=[
            pltpu.VMEM((RW,), jnp.int32),
            pltpu.VMEM((RW, CLS), jnp.float32),
            pltpu.VMEM((BAG, CLS), jnp.float32),
            pltpu.VMEM((HB,), jnp.float32),
        ],
        compiler_params=pltpu.CompilerParams(needs_layout_passes=False),
    )(_sc_body)
    return call(yt2, y_pred)


def _ep_body(sums_ref, cnt_ref, th_ref, out_ref):
    sums = sums_ref[0]
    counts = cnt_ref[0]
    for i in range(1, NW):
        sums = sums + sums_ref[i]
        counts = counts + cnt_ref[i]
    means = sums / counts
    theta_c = jnp.clip(th_ref[...], 1e-07, 1.0 - 1e-07)
    m = jnp.max(means, axis=-1, keepdims=True)
    e = jnp.exp(means - m)
    ssum = jnp.sum(e, axis=-1, keepdims=True)
    sm = e / ssum
    loss = -theta_c * jnp.log(sm + 1e-07)
    out_ref[0, 0] = jnp.sum(loss)


def kernel(y_true, y_pred, theta):
    yt2 = y_true.astype(jnp.int32).reshape(NW, RW)
    psums, phist = _sc_call(yt2, y_pred)
    out = pl.pallas_call(
        _ep_body,
        out_shape=jax.ShapeDtypeStruct((1, 1), jnp.float32),
        in_specs=[
            pl.BlockSpec(memory_space=pltpu.VMEM),
            pl.BlockSpec(memory_space=pltpu.VMEM),
            pl.BlockSpec(memory_space=pltpu.VMEM),
        ],
        out_specs=pl.BlockSpec(memory_space=pltpu.SMEM),
    )(psums, phist[:, :BAG].reshape(NW, BAG, 1), theta.reshape(BAG, 1))
    return out[0, 0]


# speedup vs baseline: 20.5340x; 6.6244x over previous
import jax
import jax.numpy as jnp
from jax.experimental import pallas as pl
from jax.experimental.pallas import tpu as pltpu

BAG = 64
CLS = 128
N = 16384


def _tc_body(yt_ref, yp_ref, th_ref, out_ref):
    ids = yt_ref[...]
    oh = (jax.lax.broadcasted_iota(jnp.int32, (BAG, 1), 0) == ids).astype(
        jnp.float32
    )
    yp = yp_ref[...]
    sums = jax.lax.dot_general(
        oh, yp, (((1,), (0,)), ((), ())), preferred_element_type=jnp.float32
    )
    counts = jnp.sum(oh, axis=1, keepdims=True)
    means = sums / counts
    theta_c = jnp.clip(th_ref[...], 1e-07, 1.0 - 1e-07)
    m = jnp.max(means, axis=-1, keepdims=True)
    e = jnp.exp(means - m)
    s = jnp.sum(e, axis=-1, keepdims=True)
    sm = e / s
    loss = -theta_c * jnp.log(sm + 1e-07)
    out_ref[0, 0] = jnp.sum(loss)


def kernel(y_true, y_pred, theta):
    yt = y_true.astype(jnp.int32).reshape(1, N)
    out = pl.pallas_call(
        _tc_body,
        out_shape=jax.ShapeDtypeStruct((1, 1), jnp.float32),
        in_specs=[
            pl.BlockSpec(memory_space=pltpu.VMEM),
            pl.BlockSpec(memory_space=pltpu.VMEM),
            pl.BlockSpec(memory_space=pltpu.VMEM),
        ],
        out_specs=pl.BlockSpec(memory_space=pltpu.SMEM),
    )(yt, y_pred, theta.reshape(BAG, 1))
    return out[0, 0]
